# Initial kernel scaffold; baseline (speedup 1.0000x reference)
import jax
import jax.numpy as jnp
from jax.experimental import pallas as pl


def kernel(x, unigram_logp, unigram_backoff, bigram_logp, bigram_found):
    raise NotImplementedError("write your pallas kernel here")


# speedup vs baseline: 48.1081x; 48.1081x over previous
import functools

import jax
import jax.numpy as jnp
from jax import lax
from jax.experimental import pallas as pl
from jax.experimental.pallas import tpu as pltpu
from jax.experimental.pallas import tpu_sc as plsc

_VOCAB = 100000
_HASH_SIZE = 4194304
_B = 4096
_L = 50
_NPAIR = _B * (_L - 1)
_NW = 32
_PER_W = _NPAIR // _NW
_VECS = _PER_W // 16


def _lm_body(prev_hbm, cur_hbm, uni_hbm, bo_hbm, bg_hbm, fnd_hbm, out_hbm,
             prev_v, cur_v, h_v, uni_v, bo_v, bg_v, fnd_v, out_v,
             s_uni, s_bo, s_bg, s_fnd):
    wid = lax.axis_index("s") * 2 + lax.axis_index("c")
    base = wid * _PER_W

    pltpu.sync_copy(prev_hbm.at[pl.ds(base, _PER_W)], prev_v)
    pltpu.sync_copy(cur_hbm.at[pl.ds(base, _PER_W)], cur_v)

    cp_uni = pltpu.async_copy(uni_hbm.at[cur_v], uni_v, s_uni)
    cp_bo = pltpu.async_copy(bo_hbm.at[prev_v], bo_v, s_bo)

    def hash_row(i, _):
        pv = prev_v[pl.ds(i * 16, 16)]
        cv = cur_v[pl.ds(i * 16, 16)]
        h_v[pl.ds(i * 16, 16)] = (pv * 1000003 + cv) & (_HASH_SIZE - 1)
        return 0

    lax.fori_loop(0, _VECS, hash_row, 0)

    cp_bg = pltpu.async_copy(bg_hbm.at[h_v], bg_v, s_bg)
    cp_fnd = pltpu.async_copy(fnd_hbm.at[h_v], fnd_v, s_fnd)

    cp_uni.wait()
    cp_bo.wait()
    cp_bg.wait()
    cp_fnd.wait()

    def blend_row(i, _):
        sl = pl.ds(i * 16, 16)
        f = fnd_v[sl]
        out_v[sl] = f * bg_v[sl] + (1.0 - f) * (bo_v[sl] + uni_v[sl])
        return 0

    lax.fori_loop(0, _VECS, blend_row, 0)

    pltpu.sync_copy(out_v, out_hbm.at[pl.ds(base, _PER_W)])


@jax.jit
def _lm(prev, cur, uni, bo, bg, fnd):
    run = pl.kernel(
        _lm_body,
        out_type=jax.ShapeDtypeStruct((_NPAIR,), jnp.float32),
        mesh=plsc.VectorSubcoreMesh(core_axis_name="c", subcore_axis_name="s"),
        scratch_types=[
            pltpu.VMEM((_PER_W,), jnp.int32),
            pltpu.VMEM((_PER_W,), jnp.int32),
            pltpu.VMEM((_PER_W,), jnp.int32),
            pltpu.VMEM((_PER_W,), jnp.float32),
            pltpu.VMEM((_PER_W,), jnp.float32),
            pltpu.VMEM((_PER_W,), jnp.float32),
            pltpu.VMEM((_PER_W,), jnp.float32),
            pltpu.VMEM((_PER_W,), jnp.float32),
            pltpu.SemaphoreType.DMA,
            pltpu.SemaphoreType.DMA,
            pltpu.SemaphoreType.DMA,
            pltpu.SemaphoreType.DMA,
        ],
    )
    return run(prev, cur, uni, bo, bg, fnd)


def kernel(x, unigram_logp, unigram_backoff, bigram_logp, bigram_found):
    x = x.astype(jnp.int32)
    prev = x[:, :-1].reshape(-1)
    cur = x[:, 1:].reshape(-1)
    out = _lm(prev, cur, unigram_logp, unigram_backoff,
              bigram_logp, bigram_found)
    return out.reshape(_B, _L - 1)


# speedup vs baseline: 48.1426x; 1.0007x over previous
import functools

import jax
import jax.numpy as jnp
from jax import lax
from jax.experimental import pallas as pl
from jax.experimental.pallas import tpu as pltpu
from jax.experimental.pallas import tpu_sc as plsc

_VOCAB = 100000
_HASH_SIZE = 4194304
_B = 4096
_L = 50
_NPAIR = _B * (_L - 1)
_NW = 32
_PER_W = _NPAIR // _NW
_VECS = _PER_W // 16


def _lm_body(prev_hbm, cur_hbm, uni_hbm, bo_hbm, bg_hbm, fnd_hbm, out_hbm,
             prev_v, cur_v, h_v, uni_v, bo_v, bg_v, fnd_v, out_v,
             s_uni, s_bo, s_bg, s_fnd):
    wid = lax.axis_index("s") * 2 + lax.axis_index("c")
    base = wid * _PER_W

    with jax.named_scope("stage"):
        pltpu.sync_copy(prev_hbm.at[pl.ds(base, _PER_W)], prev_v)
        pltpu.sync_copy(cur_hbm.at[pl.ds(base, _PER_W)], cur_v)

    cp_uni = pltpu.async_copy(uni_hbm.at[cur_v], uni_v, s_uni)
    cp_bo = pltpu.async_copy(bo_hbm.at[prev_v], bo_v, s_bo)

    def hash_row(i, _):
        pv = prev_v[pl.ds(i * 16, 16)]
        cv = cur_v[pl.ds(i * 16, 16)]
        h_v[pl.ds(i * 16, 16)] = (pv * 1000003 + cv) & (_HASH_SIZE - 1)
        return 0

    with jax.named_scope("hash"):
        lax.fori_loop(0, _VECS, hash_row, 0)

    cp_bg = pltpu.async_copy(bg_hbm.at[h_v], bg_v, s_bg)
    cp_fnd = pltpu.async_copy(fnd_hbm.at[h_v], fnd_v, s_fnd)

    with jax.named_scope("wait_uni"):
        cp_uni.wait()
        cp_bo.wait()
    with jax.named_scope("wait_bg"):
        cp_bg.wait()
        cp_fnd.wait()

    def blend_row(i, _):
        sl = pl.ds(i * 16, 16)
        f = fnd_v[sl]
        out_v[sl] = f * bg_v[sl] + (1.0 - f) * (bo_v[sl] + uni_v[sl])
        return 0

    with jax.named_scope("blend"):
        lax.fori_loop(0, _VECS, blend_row, 0)

    with jax.named_scope("out"):
        pltpu.sync_copy(out_v, out_hbm.at[pl.ds(base, _PER_W)])


@jax.jit
def _lm(prev, cur, uni, bo, bg, fnd):
    run = pl.kernel(
        _lm_body,
        out_type=jax.ShapeDtypeStruct((_NPAIR,), jnp.float32),
        mesh=plsc.VectorSubcoreMesh(core_axis_name="c", subcore_axis_name="s"),
        scratch_types=[
            pltpu.VMEM((_PER_W,), jnp.int32),
            pltpu.VMEM((_PER_W,), jnp.int32),
            pltpu.VMEM((_PER_W,), jnp.int32),
            pltpu.VMEM((_PER_W,), jnp.float32),
            pltpu.VMEM((_PER_W,), jnp.float32),
            pltpu.VMEM((_PER_W,), jnp.float32),
            pltpu.VMEM((_PER_W,), jnp.float32),
            pltpu.VMEM((_PER_W,), jnp.float32),
            pltpu.SemaphoreType.DMA,
            pltpu.SemaphoreType.DMA,
            pltpu.SemaphoreType.DMA,
            pltpu.SemaphoreType.DMA,
        ],
    )
    return run(prev, cur, uni, bo, bg, fnd)


def kernel(x, unigram_logp, unigram_backoff, bigram_logp, bigram_found):
    x = x.astype(jnp.int32)
    prev = x[:, :-1].reshape(-1)
    cur = x[:, 1:].reshape(-1)
    out = _lm(prev, cur, unigram_logp, unigram_backoff,
              bigram_logp, bigram_found)
    return out.reshape(_B, _L - 1)


# speedup vs baseline: 59.6254x; 1.2385x over previous
import jax
import jax.numpy as jnp
from jax import lax
from jax.experimental import pallas as pl
from jax.experimental.pallas import tpu as pltpu
from jax.experimental.pallas import tpu_sc as plsc

_VOCAB = 100000
_HASH_SIZE = 4194304
_B = 4096
_L = 50
_NPAIR = _B * (_L - 1)
_NW = 32
_PER_W = _NPAIR // _NW
_VECS = _PER_W // 16
_STAGE = 6256


def _lm_body(prev_hbm, cur_hbm, uni_hbm, bo_hbm, bg_hbm, fnd_hbm, out_hbm,
             prev_v, cur_v, h_v, uni_v, bo_v, bg_v, fnd_v, out_v, bounce_v,
             uni_sh, bo_sh,
             s_uni, s_bo, s_bg, s_fnd):
    sid = lax.axis_index("s")
    wid = sid * 2 + lax.axis_index("c")
    base = wid * _PER_W

    sbase = jnp.minimum(sid * _STAGE, _VOCAB - _STAGE)
    pltpu.sync_copy(uni_hbm.at[pl.ds(sbase, _STAGE)], bounce_v)
    pltpu.sync_copy(bounce_v, uni_sh.at[pl.ds(sbase, _STAGE)])
    pltpu.sync_copy(bo_hbm.at[pl.ds(sbase, _STAGE)], bounce_v)
    pltpu.sync_copy(bounce_v, bo_sh.at[pl.ds(sbase, _STAGE)])

    with jax.named_scope("stage"):
        pltpu.sync_copy(prev_hbm.at[pl.ds(base, _PER_W)], prev_v)
        pltpu.sync_copy(cur_hbm.at[pl.ds(base, _PER_W)], cur_v)

    def hash_row(i, _):
        pv = prev_v[pl.ds(i * 16, 16)]
        cv = cur_v[pl.ds(i * 16, 16)]
        h_v[pl.ds(i * 16, 16)] = (pv * 1000003 + cv) & (_HASH_SIZE - 1)
        return 0

    with jax.named_scope("hash"):
        lax.fori_loop(0, _VECS, hash_row, 0)

    cp_bg = pltpu.async_copy(bg_hbm.at[h_v], bg_v, s_bg)
    cp_fnd = pltpu.async_copy(fnd_hbm.at[h_v], fnd_v, s_fnd)

    plsc.subcore_barrier()
    cp_uni = pltpu.async_copy(uni_sh.at[cur_v], uni_v, s_uni)
    cp_bo = pltpu.async_copy(bo_sh.at[prev_v], bo_v, s_bo)

    with jax.named_scope("wait_uni"):
        cp_uni.wait()
        cp_bo.wait()
    with jax.named_scope("wait_bg"):
        cp_bg.wait()
        cp_fnd.wait()

    def blend_row(i, _):
        sl = pl.ds(i * 16, 16)
        f = fnd_v[sl]
        out_v[sl] = f * bg_v[sl] + (1.0 - f) * (bo_v[sl] + uni_v[sl])
        return 0

    with jax.named_scope("blend"):
        lax.fori_loop(0, _VECS, blend_row, 0)

    with jax.named_scope("out"):
        pltpu.sync_copy(out_v, out_hbm.at[pl.ds(base, _PER_W)])


@jax.jit
def _lm(prev, cur, uni, bo, bg, fnd):
    run = pl.kernel(
        _lm_body,
        out_type=jax.ShapeDtypeStruct((_NPAIR,), jnp.float32),
        mesh=plsc.VectorSubcoreMesh(core_axis_name="c", subcore_axis_name="s"),
        scratch_types=[
            pltpu.VMEM((_PER_W,), jnp.int32),
            pltpu.VMEM((_PER_W,), jnp.int32),
            pltpu.VMEM((_PER_W,), jnp.int32),
            pltpu.VMEM((_PER_W,), jnp.float32),
            pltpu.VMEM((_PER_W,), jnp.float32),
            pltpu.VMEM((_PER_W,), jnp.float32),
            pltpu.VMEM((_PER_W,), jnp.float32),
            pltpu.VMEM((_PER_W,), jnp.float32),
            pltpu.VMEM((_STAGE,), jnp.float32),
            pltpu.VMEM_SHARED((_VOCAB,), jnp.float32),
            pltpu.VMEM_SHARED((_VOCAB,), jnp.float32),
            pltpu.SemaphoreType.DMA,
            pltpu.SemaphoreType.DMA,
            pltpu.SemaphoreType.DMA,
            pltpu.SemaphoreType.DMA,
        ],
    )
    return run(prev, cur, uni, bo, bg, fnd)


def kernel(x, unigram_logp, unigram_backoff, bigram_logp, bigram_found):
    x = x.astype(jnp.int32)
    prev = x[:, :-1].reshape(-1)
    cur = x[:, 1:].reshape(-1)
    out = _lm(prev, cur, unigram_logp, unigram_backoff,
              bigram_logp, bigram_found)
    return out.reshape(_B, _L - 1)
